# Initial kernel scaffold; baseline (speedup 1.0000x reference)
import jax
import jax.numpy as jnp
from jax.experimental import pallas as pl


def kernel(outputs, gt):
    raise NotImplementedError("write your pallas kernel here")


# speedup vs baseline: 3.3784x; 3.3784x over previous
import functools

import jax
import jax.numpy as jnp
from jax import lax
from jax.experimental import pallas as pl
from jax.experimental.pallas import tpu as pltpu
from jax.experimental.pallas import tpu_sc as plsc

NCLS = 19
SMOOTH_V = 1e-05

NC = 2
NS = 16
L = 16
NW = NC * NS

TOTAL = 4 * 512 * 512
PER_W = TOTAL // NW
ACC = 2 * NCLS * L


def _sc_body(o_hbm, g_hbm, part_hbm, o_v, g_v, acc_v):
    wid = lax.axis_index("s") * NC + lax.axis_index("c")
    base = wid * PER_W
    pltpu.sync_copy(o_hbm.at[pl.ds(base, PER_W)], o_v)
    pltpu.sync_copy(g_hbm.at[pl.ds(base, PER_W)], g_v)

    zeros = jnp.zeros((L,), jnp.float32)
    for r in range(ACC // L):
        acc_v[pl.ds(r * L, L)] = zeros

    lane = lax.iota(jnp.int32, L)
    ones = jnp.ones((L,), jnp.float32)

    def body(i, carry):
        g = g_v[pl.ds(i * L, L)]
        o = o_v[pl.ds(i * L, L)]
        d = g.astype(jnp.float32) - o
        d2 = d * d
        idx = g * L + lane
        plsc.addupdate_scatter(acc_v, [idx], d2)
        plsc.addupdate_scatter(acc_v, [idx + NCLS * L], ones)
        return carry

    lax.fori_loop(0, PER_W // L, body, 0)
    pltpu.sync_copy(acc_v, part_hbm.at[wid])


@functools.partial(jax.jit)
def _sc_call(o_flat, g_flat):
    k = functools.partial(
        pl.kernel,
        out_type=jax.ShapeDtypeStruct((NW, ACC), jnp.float32),
        mesh=plsc.VectorSubcoreMesh(core_axis_name="c", subcore_axis_name="s"),
        compiler_params=pltpu.CompilerParams(needs_layout_passes=False),
        scratch_types=[
            pltpu.VMEM((PER_W,), jnp.float32),
            pltpu.VMEM((PER_W,), jnp.int32),
            pltpu.VMEM((ACC,), jnp.float32),
        ],
    )(_sc_body)
    return k(o_flat, g_flat)


def kernel(outputs, gt):
    o_flat = outputs.reshape(-1)
    g_flat = gt.reshape(-1)
    part = _sc_call(o_flat, g_flat)
    total = part.sum(axis=0)
    d2 = total[: NCLS * L].reshape(NCLS, L).sum(axis=-1)
    cnt = total[NCLS * L :].reshape(NCLS, L).sum(axis=-1)
    return d2 / jnp.maximum(cnt, SMOOTH_V)
